# baseline (speedup 1.0000x reference)
import functools

import jax
import jax.numpy as jnp
from jax import lax
from jax.experimental import pallas as pl
from jax.experimental.pallas import tpu as pltpu
from jax.experimental.pallas import tpu_sc as plsc

B = 16384
D = 32
NROWS = 1000000
QROWS = NROWS // 4

_info = plsc.get_sparse_core_info()
_NC = _info.num_cores
_NS = _info.num_subcores
_NW = _NC * _NS
_BPW = B // _NW
_CH = 128


def _gather_body(uq_idx, iq_idx, mfu, mfi, mlu, mli,
                 g0_out, g1_out, g2_out, g3_out,
                 uidx_v, iidx_v, r0, r1, r2, r3, s0, s1, s2, s3):
    wid = lax.axis_index("s") * _NC + lax.axis_index("c")
    base = wid * _BPW
    pltpu.sync_copy(uq_idx.at[pl.ds(base, _BPW)], uidx_v)
    pltpu.sync_copy(iq_idx.at[pl.ds(base, _BPW)], iidx_v)
    for c in range(_BPW // _CH):
        o = c * _CH
        c0 = pltpu.async_copy(mfu.at[uidx_v.at[pl.ds(o, _CH)]], r0, s0)
        c1 = pltpu.async_copy(mfi.at[iidx_v.at[pl.ds(o, _CH)]], r1, s1)
        c2 = pltpu.async_copy(mlu.at[uidx_v.at[pl.ds(o, _CH)]], r2, s2)
        c3 = pltpu.async_copy(mli.at[iidx_v.at[pl.ds(o, _CH)]], r3, s3)
        c0.wait()
        pltpu.sync_copy(r0, g0_out.at[pl.ds(base + o, _CH)])
        c1.wait()
        pltpu.sync_copy(r1, g1_out.at[pl.ds(base + o, _CH)])
        c2.wait()
        pltpu.sync_copy(r2, g2_out.at[pl.ds(base + o, _CH)])
        c3.wait()
        pltpu.sync_copy(r3, g3_out.at[pl.ds(base + o, _CH)])


_gather = pl.kernel(
    _gather_body,
    out_type=(
        jax.ShapeDtypeStruct((B, 128), jnp.float32),
        jax.ShapeDtypeStruct((B, 128), jnp.float32),
        jax.ShapeDtypeStruct((B, 128), jnp.float32),
        jax.ShapeDtypeStruct((B, 128), jnp.float32),
    ),
    mesh=plsc.VectorSubcoreMesh(core_axis_name="c", subcore_axis_name="s"),
    scratch_types=[
        pltpu.VMEM((_BPW,), jnp.int32),
        pltpu.VMEM((_BPW,), jnp.int32),
        pltpu.VMEM((_CH, 128), jnp.float32),
        pltpu.VMEM((_CH, 128), jnp.float32),
        pltpu.VMEM((_CH, 128), jnp.float32),
        pltpu.VMEM((_CH, 128), jnp.float32),
        pltpu.SemaphoreType.DMA,
        pltpu.SemaphoreType.DMA,
        pltpu.SemaphoreType.DMA,
        pltpu.SemaphoreType.DMA,
    ],
)


def _pick(g, p32):
    x = jnp.where(p32 < 2,
                  jnp.where(p32 == 0, g[:, 0:32], g[:, 32:64]),
                  jnp.where(p32 == 2, g[:, 64:96], g[:, 96:128]))
    return x


def _mlp_body(g_mfu, g_mfi, g_mlu, g_mli, pu32, pi32,
              W1, b1, W2, b2, Wp, bp,
              out, umf_o, imf_o, umlp_o, imlp_o):
    umf = _pick(g_mfu, pu32[...])
    imf = _pick(g_mfi, pi32[...])
    umlp = _pick(g_mlu, pu32[...])
    imlp = _pick(g_mli, pi32[...])
    umf_o[...] = umf
    imf_o[...] = imf
    umlp_o[...] = umlp
    imlp_o[...] = imlp
    mf = umf * imf
    mlp = jnp.concatenate([umlp, imlp], axis=1)
    h = lax.dot_general(mlp, W1[...], (((1,), (1,)), ((), ())),
                        preferred_element_type=jnp.float32) + b1[...]
    h = jnp.maximum(h, 0.0)
    h = lax.dot_general(h, W2[...], (((1,), (1,)), ((), ())),
                        preferred_element_type=jnp.float32) + b2[...]
    h = jnp.maximum(h, 0.0)
    pv = jnp.concatenate([mf, h], axis=1)
    logit = jnp.sum(pv * Wp[...], axis=1) + bp[0, 0]
    out[...] = jax.nn.sigmoid(logit)


_BLK = 2048


def _mlp(g_mfu, g_mfi, g_mlu, g_mli, pu32, pi32, W1, b1, W2, b2, Wp, bp):
    nb = B // _BLK
    row_spec = pl.BlockSpec((_BLK, 128), lambda i: (i, 0))
    par_spec = pl.BlockSpec((_BLK, D), lambda i: (i, 0))
    lat_spec = pl.BlockSpec((_BLK, D), lambda i: (i, 0))
    return pl.pallas_call(
        _mlp_body,
        grid=(nb,),
        in_specs=[
            row_spec, row_spec, row_spec, row_spec,
            par_spec, par_spec,
            pl.BlockSpec((64, 64), lambda i: (0, 0)),
            pl.BlockSpec((1, 64), lambda i: (0, 0)),
            pl.BlockSpec((32, 64), lambda i: (0, 0)),
            pl.BlockSpec((1, 32), lambda i: (0, 0)),
            pl.BlockSpec((1, 64), lambda i: (0, 0)),
            pl.BlockSpec((1, 1), lambda i: (0, 0)),
        ],
        out_specs=[
            pl.BlockSpec((_BLK,), lambda i: (i,)),
            lat_spec, lat_spec, lat_spec, lat_spec,
        ],
        out_shape=[
            jax.ShapeDtypeStruct((B,), jnp.float32),
            jax.ShapeDtypeStruct((B, D), jnp.float32),
            jax.ShapeDtypeStruct((B, D), jnp.float32),
            jax.ShapeDtypeStruct((B, D), jnp.float32),
            jax.ShapeDtypeStruct((B, D), jnp.float32),
        ],
    )(g_mfu, g_mfi, g_mlu, g_mli, pu32, pi32,
      W1, b1.reshape(1, 64), W2, b2.reshape(1, 32), Wp, bp.reshape(1, 1))


def kernel(user_indices, item_indices, mf_user_emb, mf_item_emb,
           mlp_user_emb, mlp_item_emb, W1, b1, W2, b2, Wp, bp):
    ui = user_indices.astype(jnp.int32)
    ii = item_indices.astype(jnp.int32)
    uq = ui >> 2
    iq = ii >> 2
    pu32 = jnp.broadcast_to((ui & 3)[:, None], (B, D))
    pi32 = jnp.broadcast_to((ii & 3)[:, None], (B, D))
    def q128(t):
        return jnp.concatenate([t[0::4], t[1::4], t[2::4], t[3::4]], axis=1)

    g_mfu, g_mfi, g_mlu, g_mli = _gather(
        uq, iq,
        q128(mf_user_emb), q128(mf_item_emb),
        q128(mlp_user_emb), q128(mlp_item_emb))
    pred, umf, imf, umlp, imlp = _mlp(
        g_mfu, g_mfi, g_mlu, g_mli, pu32, pi32, W1, b1, W2, b2, Wp, bp)
    user_repr = jnp.concatenate([umf, umlp], axis=0)
    item_repr = jnp.concatenate([imf, imlp], axis=0)
    return (pred, user_repr, item_repr)


# speedup vs baseline: 1.7599x; 1.7599x over previous
import functools

import jax
import jax.numpy as jnp
from jax import lax
from jax.experimental import pallas as pl
from jax.experimental.pallas import tpu as pltpu
from jax.experimental.pallas import tpu_sc as plsc

B = 16384
D = 32

_info = plsc.get_sparse_core_info()
_NC = _info.num_cores
_NS = _info.num_subcores
_NW = _NC * _NS
_BPW = B // _NW


def _gather_body(u_idx, i_idx, mfuT, mfiT, mluT, mliT, uT_out, iT_out,
                 u_vm, i_vm, l0, l1, l2, l3, s0, s1, s2, s3):
    wid = lax.axis_index("s") * _NC + lax.axis_index("c")
    base = wid * _BPW
    pltpu.sync_copy(u_idx.at[pl.ds(base, _BPW)], u_vm)
    pltpu.sync_copy(i_idx.at[pl.ds(base, _BPW)], i_vm)
    tbls = (mfuT, mfiT, mluT, mliT)
    idxs = (u_vm, i_vm, u_vm, i_vm)
    lats = (l0, l1, l2, l3)
    sems = (s0, s1, s2, s3)
    copies = []
    for d in range(D):
        for t in range(4):
            copies.append(pltpu.async_copy(
                tbls[t].at[d].at[idxs[t]], lats[t].at[d], sems[t]))
    for c in copies:
        c.wait()
    pltpu.sync_copy(l0, uT_out.at[:, pl.ds(base, _BPW)])
    pltpu.sync_copy(l2, uT_out.at[:, pl.ds(B + base, _BPW)])
    pltpu.sync_copy(l1, iT_out.at[:, pl.ds(base, _BPW)])
    pltpu.sync_copy(l3, iT_out.at[:, pl.ds(B + base, _BPW)])


_gather = pl.kernel(
    _gather_body,
    out_type=(
        jax.ShapeDtypeStruct((D, 2 * B), jnp.float32),
        jax.ShapeDtypeStruct((D, 2 * B), jnp.float32),
    ),
    mesh=plsc.VectorSubcoreMesh(core_axis_name="c", subcore_axis_name="s"),
    compiler_params=pltpu.CompilerParams(use_tc_tiling_on_sc=False),
    scratch_types=[
        pltpu.VMEM((_BPW,), jnp.int32),
        pltpu.VMEM((_BPW,), jnp.int32),
        pltpu.VMEM((D, _BPW), jnp.float32),
        pltpu.VMEM((D, _BPW), jnp.float32),
        pltpu.VMEM((D, _BPW), jnp.float32),
        pltpu.VMEM((D, _BPW), jnp.float32),
        pltpu.SemaphoreType.DMA,
        pltpu.SemaphoreType.DMA,
        pltpu.SemaphoreType.DMA,
        pltpu.SemaphoreType.DMA,
    ],
)


def _mlp_body(umfT, imfT, umlpT, imlpT, W1, b1b, W2, b2b, Wp, bp, out):
    mfT = umfT[...] * imfT[...]
    xT = jnp.concatenate([umlpT[...], imlpT[...]], axis=0)
    h = lax.dot_general(W1[...], xT, (((1,), (0,)), ((), ())),
                        preferred_element_type=jnp.float32) + b1b[...]
    h = jnp.maximum(h, 0.0)
    h = lax.dot_general(W2[...], h, (((1,), (0,)), ((), ())),
                        preferred_element_type=jnp.float32) + b2b[...]
    h = jnp.maximum(h, 0.0)
    pvT = jnp.concatenate([mfT, h], axis=0)
    logit = lax.dot_general(Wp[...], pvT, (((1,), (0,)), ((), ())),
                            preferred_element_type=jnp.float32) + bp[0, 0]
    out[...] = jax.nn.sigmoid(logit)


_BLK = 2048


def _mlp(uT, iT, W1, b1, W2, b2, Wp, bp):
    nb = B // _BLK
    half = B // _BLK
    return pl.pallas_call(
        _mlp_body,
        grid=(nb,),
        in_specs=[
            pl.BlockSpec((D, _BLK), lambda i: (0, i)),
            pl.BlockSpec((D, _BLK), lambda i: (0, i)),
            pl.BlockSpec((D, _BLK), lambda i: (0, i + half)),
            pl.BlockSpec((D, _BLK), lambda i: (0, i + half)),
            pl.BlockSpec((64, 64), lambda i: (0, 0)),
            pl.BlockSpec((64, _BLK), lambda i: (0, 0)),
            pl.BlockSpec((32, 64), lambda i: (0, 0)),
            pl.BlockSpec((32, _BLK), lambda i: (0, 0)),
            pl.BlockSpec((1, 64), lambda i: (0, 0)),
            pl.BlockSpec((1, 1), lambda i: (0, 0)),
        ],
        out_specs=pl.BlockSpec((1, _BLK), lambda i: (0, i)),
        out_shape=jax.ShapeDtypeStruct((1, B), jnp.float32),
    )(uT, iT, uT, iT,
      W1, jnp.broadcast_to(b1[:, None], (64, _BLK)),
      W2, jnp.broadcast_to(b2[:, None], (32, _BLK)),
      Wp, bp.reshape(1, 1))


def kernel(user_indices, item_indices, mf_user_emb, mf_item_emb,
           mlp_user_emb, mlp_item_emb, W1, b1, W2, b2, Wp, bp):
    uT, iT = _gather(
        user_indices.astype(jnp.int32), item_indices.astype(jnp.int32),
        mf_user_emb.T, mf_item_emb.T, mlp_user_emb.T, mlp_item_emb.T)
    predT = _mlp(uT, iT, W1, b1, W2, b2, Wp, bp)
    return (predT.reshape(B), uT.T, iT.T)


# speedup vs baseline: 10.4777x; 5.9535x over previous
import functools

import jax
import jax.numpy as jnp
from jax import lax
from jax.experimental import pallas as pl
from jax.experimental.pallas import tpu as pltpu
from jax.experimental.pallas import tpu_sc as plsc

B = 16384
D = 32

_info = plsc.get_sparse_core_info()
_NC = _info.num_cores
_NS = _info.num_subcores
_NW = _NC * _NS
_BPW = B // _NW


def _gather_body(u_idx, i_idx, mfu, mfi, mlu, mli, user_out, item_out,
                 uidx_v, iidx_v, r0, r1, r2, r3, s0, s1, s2, s3):
    wid = lax.axis_index("s") * _NC + lax.axis_index("c")
    base = wid * _BPW
    pltpu.sync_copy(u_idx.at[pl.ds(base, _BPW)], uidx_v)
    pltpu.sync_copy(i_idx.at[pl.ds(base, _BPW)], iidx_v)
    c0 = pltpu.async_copy(mfu.at[uidx_v], r0, s0)
    c1 = pltpu.async_copy(mfi.at[iidx_v], r1, s1)
    c2 = pltpu.async_copy(mlu.at[uidx_v], r2, s2)
    c3 = pltpu.async_copy(mli.at[iidx_v], r3, s3)
    c0.wait()
    pltpu.sync_copy(r0, user_out.at[pl.ds(base, _BPW)])
    c1.wait()
    pltpu.sync_copy(r1, item_out.at[pl.ds(base, _BPW)])
    c2.wait()
    pltpu.sync_copy(r2, user_out.at[pl.ds(B + base, _BPW)])
    c3.wait()
    pltpu.sync_copy(r3, item_out.at[pl.ds(B + base, _BPW)])


_gather = pl.kernel(
    _gather_body,
    out_type=(
        jax.ShapeDtypeStruct((2 * B, D), jnp.float32),
        jax.ShapeDtypeStruct((2 * B, D), jnp.float32),
    ),
    mesh=plsc.VectorSubcoreMesh(core_axis_name="c", subcore_axis_name="s"),
    compiler_params=pltpu.CompilerParams(use_tc_tiling_on_sc=False),
    scratch_types=[
        pltpu.VMEM((_BPW,), jnp.int32),
        pltpu.VMEM((_BPW,), jnp.int32),
        pltpu.VMEM((_BPW, D), jnp.float32),
        pltpu.VMEM((_BPW, D), jnp.float32),
        pltpu.VMEM((_BPW, D), jnp.float32),
        pltpu.VMEM((_BPW, D), jnp.float32),
        pltpu.SemaphoreType.DMA,
        pltpu.SemaphoreType.DMA,
        pltpu.SemaphoreType.DMA,
        pltpu.SemaphoreType.DMA,
    ],
)


def _mlp_body(umf, imf, umlp, imlp, W1, b1, W2, b2, Wp, bp, out):
    mf = umf[...] * imf[...]
    mlp = jnp.concatenate([umlp[...], imlp[...]], axis=1)
    h = lax.dot_general(mlp, W1[...], (((1,), (1,)), ((), ())),
                        preferred_element_type=jnp.float32) + b1[...]
    h = jnp.maximum(h, 0.0)
    h = lax.dot_general(h, W2[...], (((1,), (1,)), ((), ())),
                        preferred_element_type=jnp.float32) + b2[...]
    h = jnp.maximum(h, 0.0)
    pv = jnp.concatenate([mf, h], axis=1)
    logit = jnp.sum(pv * Wp[...], axis=1) + bp[0, 0]
    out[...] = jax.nn.sigmoid(logit)


_BLK = 2048


def _mlp(user_repr, item_repr, W1, b1, W2, b2, Wp, bp):
    nb = B // _BLK
    half = B // _BLK
    return pl.pallas_call(
        _mlp_body,
        grid=(nb,),
        in_specs=[
            pl.BlockSpec((_BLK, D), lambda i: (i, 0)),
            pl.BlockSpec((_BLK, D), lambda i: (i, 0)),
            pl.BlockSpec((_BLK, D), lambda i: (i + half, 0)),
            pl.BlockSpec((_BLK, D), lambda i: (i + half, 0)),
            pl.BlockSpec((64, 64), lambda i: (0, 0)),
            pl.BlockSpec((1, 64), lambda i: (0, 0)),
            pl.BlockSpec((32, 64), lambda i: (0, 0)),
            pl.BlockSpec((1, 32), lambda i: (0, 0)),
            pl.BlockSpec((1, 64), lambda i: (0, 0)),
            pl.BlockSpec((1, 1), lambda i: (0, 0)),
        ],
        out_specs=pl.BlockSpec((_BLK,), lambda i: (i,)),
        out_shape=jax.ShapeDtypeStruct((B,), jnp.float32),
    )(user_repr, item_repr, user_repr, item_repr,
      W1, b1.reshape(1, 64), W2, b2.reshape(1, 32), Wp, bp.reshape(1, 1))


def kernel(user_indices, item_indices, mf_user_emb, mf_item_emb,
           mlp_user_emb, mlp_item_emb, W1, b1, W2, b2, Wp, bp):
    user_repr, item_repr = _gather(
        user_indices.astype(jnp.int32), item_indices.astype(jnp.int32),
        mf_user_emb, mf_item_emb, mlp_user_emb, mlp_item_emb)
    prediction = _mlp(user_repr, item_repr, W1, b1, W2, b2, Wp, bp)
    return (prediction, user_repr, item_repr)


# speedup vs baseline: 12.1684x; 1.1614x over previous
import functools

import jax
import jax.numpy as jnp
from jax import lax
from jax.experimental import pallas as pl
from jax.experimental.pallas import tpu as pltpu
from jax.experimental.pallas import tpu_sc as plsc

B = 16384
D = 32
NROWS = 1000000
HROWS = NROWS // 2

_info = plsc.get_sparse_core_info()
_NC = _info.num_cores
_NS = _info.num_subcores
_NW = _NC * _NS
_BPW = B // _NW
_CH = 128


def _gather_body(uh_idx, ih_idx, tu, ti, gu_out, gi_out,
                 uidx_v, iidx_v, r0, r1, s0, s1):
    wid = lax.axis_index("s") * _NC + lax.axis_index("c")
    base = wid * _BPW
    pltpu.sync_copy(uh_idx.at[pl.ds(base, _BPW)], uidx_v)
    pltpu.sync_copy(ih_idx.at[pl.ds(base, _BPW)], iidx_v)
    for c in range(_BPW // _CH):
        o = c * _CH
        c0 = pltpu.async_copy(tu.at[uidx_v.at[pl.ds(o, _CH)]], r0, s0)
        c1 = pltpu.async_copy(ti.at[iidx_v.at[pl.ds(o, _CH)]], r1, s1)
        c0.wait()
        pltpu.sync_copy(r0, gu_out.at[pl.ds(base + o, _CH)])
        c1.wait()
        pltpu.sync_copy(r1, gi_out.at[pl.ds(base + o, _CH)])


_gather = pl.kernel(
    _gather_body,
    out_type=(
        jax.ShapeDtypeStruct((B, 128), jnp.float32),
        jax.ShapeDtypeStruct((B, 128), jnp.float32),
    ),
    mesh=plsc.VectorSubcoreMesh(core_axis_name="c", subcore_axis_name="s"),
    scratch_types=[
        pltpu.VMEM((_BPW,), jnp.int32),
        pltpu.VMEM((_BPW,), jnp.int32),
        pltpu.VMEM((_CH, 128), jnp.float32),
        pltpu.VMEM((_CH, 128), jnp.float32),
        pltpu.SemaphoreType.DMA,
        pltpu.SemaphoreType.DMA,
    ],
)


def _pick(g, p32):
    lo = jnp.where(p32 == 0, g[:, 0:32], g[:, 64:96])
    hi = jnp.where(p32 == 0, g[:, 32:64], g[:, 96:128])
    return lo, hi


def _mlp_body(g_u, g_i, pu32, pi32, W1, b1, W2, b2, Wp, bp,
              out, umf_o, imf_o, umlp_o, imlp_o):
    umf, umlp = _pick(g_u, pu32[...])
    imf, imlp = _pick(g_i, pi32[...])
    umf_o[...] = umf
    imf_o[...] = imf
    umlp_o[...] = umlp
    imlp_o[...] = imlp
    mf = umf * imf
    mlp = jnp.concatenate([umlp, imlp], axis=1)
    h = lax.dot_general(mlp, W1[...], (((1,), (1,)), ((), ())),
                        preferred_element_type=jnp.float32) + b1[...]
    h = jnp.maximum(h, 0.0)
    h = lax.dot_general(h, W2[...], (((1,), (1,)), ((), ())),
                        preferred_element_type=jnp.float32) + b2[...]
    h = jnp.maximum(h, 0.0)
    pv = jnp.concatenate([mf, h], axis=1)
    logit = jnp.sum(pv * Wp[...], axis=1) + bp[0, 0]
    out[...] = jax.nn.sigmoid(logit)


_BLK = 2048


def _mlp(g_u, g_i, pu32, pi32, W1, b1, W2, b2, Wp, bp):
    nb = B // _BLK
    row_spec = pl.BlockSpec((_BLK, 128), lambda i: (i, 0))
    par_spec = pl.BlockSpec((_BLK, D), lambda i: (i, 0))
    lat_spec = pl.BlockSpec((_BLK, D), lambda i: (i, 0))
    return pl.pallas_call(
        _mlp_body,
        grid=(nb,),
        in_specs=[
            row_spec, row_spec,
            par_spec, par_spec,
            pl.BlockSpec((64, 64), lambda i: (0, 0)),
            pl.BlockSpec((1, 64), lambda i: (0, 0)),
            pl.BlockSpec((32, 64), lambda i: (0, 0)),
            pl.BlockSpec((1, 32), lambda i: (0, 0)),
            pl.BlockSpec((1, 64), lambda i: (0, 0)),
            pl.BlockSpec((1, 1), lambda i: (0, 0)),
        ],
        out_specs=[
            pl.BlockSpec((_BLK,), lambda i: (i,)),
            lat_spec, lat_spec, lat_spec, lat_spec,
        ],
        out_shape=[
            jax.ShapeDtypeStruct((B,), jnp.float32),
            jax.ShapeDtypeStruct((B, D), jnp.float32),
            jax.ShapeDtypeStruct((B, D), jnp.float32),
            jax.ShapeDtypeStruct((B, D), jnp.float32),
            jax.ShapeDtypeStruct((B, D), jnp.float32),
        ],
    )(g_u, g_i, pu32, pi32,
      W1, b1.reshape(1, 64), W2, b2.reshape(1, 32), Wp, bp.reshape(1, 1))


def kernel(user_indices, item_indices, mf_user_emb, mf_item_emb,
           mlp_user_emb, mlp_item_emb, W1, b1, W2, b2, Wp, bp):
    ui = user_indices.astype(jnp.int32)
    ii = item_indices.astype(jnp.int32)
    uh = ui >> 1
    ih = ii >> 1
    pu32 = jnp.broadcast_to((ui & 1)[:, None], (B, D))
    pi32 = jnp.broadcast_to((ii & 1)[:, None], (B, D))
    tu = jnp.concatenate([mf_user_emb, mlp_user_emb], axis=1).reshape(HROWS, 128)
    ti = jnp.concatenate([mf_item_emb, mlp_item_emb], axis=1).reshape(HROWS, 128)
    g_u, g_i = _gather(uh, ih, tu, ti)
    pred, umf, imf, umlp, imlp = _mlp(
        g_u, g_i, pu32, pi32, W1, b1, W2, b2, Wp, bp)
    user_repr = jnp.concatenate([umf, umlp], axis=0)
    item_repr = jnp.concatenate([imf, imlp], axis=0)
    return (pred, user_repr, item_repr)


# speedup vs baseline: 12.3204x; 1.0125x over previous
import functools

import jax
import jax.numpy as jnp
from jax import lax
from jax.experimental import pallas as pl
from jax.experimental.pallas import tpu as pltpu
from jax.experimental.pallas import tpu_sc as plsc

B = 16384
D = 32
NROWS = 1000000

_info = plsc.get_sparse_core_info()
_NC = _info.num_cores
_NS = _info.num_subcores
_NW = _NC * _NS
_BPW = B // _NW
_CH = 128


def _gather_body(u_idx, i_idx, tbl, gu_out, gi_out,
                 uidx_v, iidx_v, r0, r1, s0, s1):
    wid = lax.axis_index("s") * _NC + lax.axis_index("c")
    base = wid * _BPW
    pltpu.sync_copy(u_idx.at[pl.ds(base, _BPW)], uidx_v)
    pltpu.sync_copy(i_idx.at[pl.ds(base, _BPW)], iidx_v)
    for c in range(_BPW // _CH):
        o = c * _CH
        c0 = pltpu.async_copy(tbl.at[uidx_v.at[pl.ds(o, _CH)]], r0, s0)
        c1 = pltpu.async_copy(tbl.at[iidx_v.at[pl.ds(o, _CH)]], r1, s1)
        c0.wait()
        pltpu.sync_copy(r0, gu_out.at[pl.ds(base + o, _CH)])
        c1.wait()
        pltpu.sync_copy(r1, gi_out.at[pl.ds(base + o, _CH)])


_gather = pl.kernel(
    _gather_body,
    out_type=(
        jax.ShapeDtypeStruct((B, 128), jnp.float32),
        jax.ShapeDtypeStruct((B, 128), jnp.float32),
    ),
    mesh=plsc.VectorSubcoreMesh(core_axis_name="c", subcore_axis_name="s"),
    scratch_types=[
        pltpu.VMEM((_BPW,), jnp.int32),
        pltpu.VMEM((_BPW,), jnp.int32),
        pltpu.VMEM((_CH, 128), jnp.float32),
        pltpu.VMEM((_CH, 128), jnp.float32),
        pltpu.SemaphoreType.DMA,
        pltpu.SemaphoreType.DMA,
    ],
)


def _mlp_body(g_u, g_i, W1, b1, W2, b2, Wp, bp,
              out, umf_o, imf_o, umlp_o, imlp_o):
    umf = g_u[:, 0:32]
    umlp = g_u[:, 32:64]
    imf = g_i[:, 64:96]
    imlp = g_i[:, 96:128]
    umf_o[...] = umf
    imf_o[...] = imf
    umlp_o[...] = umlp
    imlp_o[...] = imlp
    mf = umf * imf
    mlp = jnp.concatenate([umlp, imlp], axis=1)
    h = lax.dot_general(mlp, W1[...], (((1,), (1,)), ((), ())),
                        preferred_element_type=jnp.float32) + b1[...]
    h = jnp.maximum(h, 0.0)
    h = lax.dot_general(h, W2[...], (((1,), (1,)), ((), ())),
                        preferred_element_type=jnp.float32) + b2[...]
    h = jnp.maximum(h, 0.0)
    pv = jnp.concatenate([mf, h], axis=1)
    logit = jnp.sum(pv * Wp[...], axis=1) + bp[0, 0]
    out[...] = jax.nn.sigmoid(logit)


_BLK = 2048


def _mlp(g_u, g_i, W1, b1, W2, b2, Wp, bp):
    nb = B // _BLK
    row_spec = pl.BlockSpec((_BLK, 128), lambda i: (i, 0))
    lat_spec = pl.BlockSpec((_BLK, D), lambda i: (i, 0))
    return pl.pallas_call(
        _mlp_body,
        grid=(nb,),
        in_specs=[
            row_spec, row_spec,
            pl.BlockSpec((64, 64), lambda i: (0, 0)),
            pl.BlockSpec((1, 64), lambda i: (0, 0)),
            pl.BlockSpec((32, 64), lambda i: (0, 0)),
            pl.BlockSpec((1, 32), lambda i: (0, 0)),
            pl.BlockSpec((1, 64), lambda i: (0, 0)),
            pl.BlockSpec((1, 1), lambda i: (0, 0)),
        ],
        out_specs=[
            pl.BlockSpec((_BLK,), lambda i: (i,)),
            lat_spec, lat_spec, lat_spec, lat_spec,
        ],
        out_shape=[
            jax.ShapeDtypeStruct((B,), jnp.float32),
            jax.ShapeDtypeStruct((B, D), jnp.float32),
            jax.ShapeDtypeStruct((B, D), jnp.float32),
            jax.ShapeDtypeStruct((B, D), jnp.float32),
            jax.ShapeDtypeStruct((B, D), jnp.float32),
        ],
    )(g_u, g_i,
      W1, b1.reshape(1, 64), W2, b2.reshape(1, 32), Wp, bp.reshape(1, 1))


def kernel(user_indices, item_indices, mf_user_emb, mf_item_emb,
           mlp_user_emb, mlp_item_emb, W1, b1, W2, b2, Wp, bp):
    ui = user_indices.astype(jnp.int32)
    ii = item_indices.astype(jnp.int32)
    t128 = jnp.concatenate(
        [mf_user_emb, mlp_user_emb, mf_item_emb, mlp_item_emb], axis=1)
    g_u, g_i = _gather(ui, ii, t128)
    pred, umf, imf, umlp, imlp = _mlp(g_u, g_i, W1, b1, W2, b2, Wp, bp)
    user_repr = jnp.concatenate([umf, umlp], axis=0)
    item_repr = jnp.concatenate([imf, imlp], axis=0)
    return (pred, user_repr, item_repr)
